# baseline (speedup 1.0000x reference)
import functools

import jax
import jax.numpy as jnp
import numpy as np
from jax import lax
from jax.experimental import pallas as pl
from jax.experimental.pallas import tpu as pltpu
from jax.experimental.pallas import tpu_sc as plsc

N = 10000
E = 160000
DIN = 256
DH = 32
DO = 7

NC = 2
NS = 16
NW = NC * NS
SUB = 128
EPW = 5120
NSUB = EPW // SUB
G = 5
NG = NSUB // G
E_PAD = NW * EPW
RPT = 624
TAIL = N - NS * RPT


def _threefry2x32(k0, k1, x0, x1):
    R0 = (13, 15, 26, 6)
    R1 = (17, 29, 16, 24)
    ks = (k0, k1, np.uint32(k0 ^ k1 ^ np.uint32(0x1BD11BDA)))
    x0 = (x0 + ks[0]).astype(np.uint32)
    x1 = (x1 + ks[1]).astype(np.uint32)
    for d in range(5):
        for r in (R0 if d % 2 == 0 else R1):
            x0 = (x0 + x1).astype(np.uint32)
            x1 = ((x1 << np.uint32(r)) | (x1 >> np.uint32(32 - r))).astype(np.uint32)
            x1 = x1 ^ x0
        x0 = (x0 + ks[(d + 1) % 3]).astype(np.uint32)
        x1 = (x1 + ks[(d + 2) % 3] + np.uint32(d + 1)).astype(np.uint32)
    return x0, x1


def _bernoulli_mask(seed, shape):
    n = int(np.prod(shape))
    x0, x1 = _threefry2x32(np.uint32(0), np.uint32(seed),
                           np.zeros(n, np.uint32),
                           np.arange(n, dtype=np.uint32))
    bits = x0 ^ x1
    floats = ((bits >> np.uint32(9)) | np.uint32(0x3F800000)).view(np.float32) - 1.0
    return (floats < 0.5).reshape(shape)


_MASK2 = _bernoulli_mask(42, (N, DH)).astype(np.float32) * 2.0


def _make_spmm(D):
    mesh = plsc.VectorSubcoreMesh(core_axis_name="c", subcore_axis_name="s")

    @functools.partial(
        pl.kernel,
        mesh=mesh,
        compiler_params=pltpu.CompilerParams(use_tc_tiling_on_sc=False),
        out_type=jax.ShapeDtypeStruct((NC, N, D), jnp.float32),
        scratch_types=[
            pltpu.VMEM((NSUB, SUB), jnp.int32),
            pltpu.VMEM((NSUB, SUB), jnp.int32),
            pltpu.VMEM((EPW,), jnp.float32),
            pltpu.VMEM((2, G * SUB, D), jnp.float32),
            pltpu.VMEM_SHARED((N, D), jnp.float32),
            pltpu.VMEM_SHARED((N, D), jnp.float32),
            pltpu.SemaphoreType.DMA,
        ],
    )
    def spmm(src_hbm, dst_hbm, w_hbm, x_hbm, zeros_hbm, out_hbm,
             src_v, dst_v, w_v, rows_v, acc_sh, x_sh, sem):
        cid = lax.axis_index("c")
        sid = lax.axis_index("s")
        wid = cid * NS + sid

        stage = [
            pltpu.async_copy(zeros_hbm.at[pl.ds(sid * RPT, RPT)],
                             acc_sh.at[pl.ds(sid * RPT, RPT)], sem),
            pltpu.async_copy(x_hbm.at[pl.ds(sid * RPT, RPT)],
                             x_sh.at[pl.ds(sid * RPT, RPT)], sem),
            pltpu.async_copy(src_hbm.at[pl.ds(wid * NSUB, NSUB)], src_v, sem),
            pltpu.async_copy(dst_hbm.at[pl.ds(wid * NSUB, NSUB)], dst_v, sem),
            pltpu.async_copy(w_hbm.at[pl.ds(wid * EPW, EPW)], w_v, sem),
        ]

        @pl.when(sid == NS - 1)
        def _():
            pltpu.async_copy(zeros_hbm.at[pl.ds(NS * RPT, TAIL)],
                             acc_sh.at[pl.ds(NS * RPT, TAIL)], sem).wait()
            pltpu.async_copy(x_hbm.at[pl.ds(NS * RPT, TAIL)],
                             x_sh.at[pl.ds(NS * RPT, TAIL)], sem).wait()

        for c in stage:
            c.wait()

        plsc.subcore_barrier()

        def fire(g, b):
            for k in range(G):
                pltpu.async_copy(
                    x_sh.at[src_v.at[g * G + k]],
                    rows_v.at[b, pl.ds(k * SUB, SUB)], sem)

        def drain(g, b):
            for k in range(G):
                pltpu.make_async_copy(
                    x_sh.at[src_v.at[g * G + k]],
                    rows_v.at[b, pl.ds(k * SUB, SUB)], sem).wait()

        def process(g, b):
            for k in range(G):
                j = g * G + k

                def scale(q, c, _k=k):
                    w16 = w_v[pl.ds((g * G + _k) * SUB + q * 16, 16)]
                    for e in range(16):
                        wb = jnp.full((16,), w16[e], jnp.float32)
                        r = _k * SUB + q * 16 + e
                        for d in range(D // 16):
                            v = rows_v[b, r, pl.ds(d * 16, 16)]
                            rows_v[b, r, pl.ds(d * 16, 16)] = v * wb
                    return c

                lax.fori_loop(0, SUB // 16, scale, 0, unroll=4)
                pltpu.sync_copy(rows_v.at[b, pl.ds(k * SUB, SUB)],
                                acc_sh.at[dst_v.at[j]], add=True)

        fire(0, 0)

        def pipe(t, carry):
            g0 = 2 * t
            fire(g0 + 1, 1)
            drain(g0, 0)
            process(g0, 0)

            @pl.when(g0 + 2 < NG)
            def _():
                fire(g0 + 2, 0)

            drain(g0 + 1, 1)
            process(g0 + 1, 1)
            return carry

        lax.fori_loop(0, NG // 2, pipe, 0)

        plsc.subcore_barrier()

        pltpu.sync_copy(acc_sh.at[pl.ds(sid * RPT, RPT)],
                        out_hbm.at[cid, pl.ds(sid * RPT, RPT)])

        @pl.when(sid == NS - 1)
        def _():
            pltpu.sync_copy(acc_sh.at[pl.ds(NS * RPT, TAIL)],
                            out_hbm.at[cid, pl.ds(NS * RPT, TAIL)])

    return spmm


_spmm32 = _make_spmm(DH)
_spmm16 = _make_spmm(16)


def _mm1_body(x_ref, w_ref, o_ref):
    o_ref[...] = jnp.dot(x_ref[...], w_ref[...],
                         preferred_element_type=jnp.float32)


def _tc1(x, w):
    bm = 2000
    return pl.pallas_call(
        _mm1_body,
        grid=(N // bm,),
        in_specs=[
            pl.BlockSpec((bm, DIN), lambda i: (i, 0)),
            pl.BlockSpec((DIN, DH), lambda i: (0, 0)),
        ],
        out_specs=pl.BlockSpec((bm, DH), lambda i: (i, 0)),
        out_shape=jax.ShapeDtypeStruct((N, DH), jnp.float32),
    )(x, w)


def _tc2_body(p_ref, b1_ref, m_ref, w2_ref, o_ref):
    h = p_ref[0] + p_ref[1] + b1_ref[...]
    h = jnp.maximum(h, 0.0) * m_ref[...]
    o_ref[...] = jnp.dot(h, w2_ref[...], preferred_element_type=jnp.float32)


def _tc2(p, b1, mask2, w2p):
    bm = 2000
    return pl.pallas_call(
        _tc2_body,
        grid=(N // bm,),
        in_specs=[
            pl.BlockSpec((NC, bm, DH), lambda i: (0, i, 0)),
            pl.BlockSpec((1, DH), lambda i: (0, 0)),
            pl.BlockSpec((bm, DH), lambda i: (i, 0)),
            pl.BlockSpec((DH, 16), lambda i: (0, 0)),
        ],
        out_specs=pl.BlockSpec((bm, 16), lambda i: (i, 0)),
        out_shape=jax.ShapeDtypeStruct((N, 16), jnp.float32),
    )(p, b1, mask2, w2p)


def _tc3_body(p_ref, b2_ref, o_ref):
    s = p_ref[0] + p_ref[1] + b2_ref[...]
    o_ref[...] = s[:, :DO]


def _tc3(p, b2p):
    bm = 2000
    return pl.pallas_call(
        _tc3_body,
        grid=(N // bm,),
        in_specs=[
            pl.BlockSpec((NC, bm, 16), lambda i: (0, i, 0)),
            pl.BlockSpec((1, 16), lambda i: (0, 0)),
        ],
        out_specs=pl.BlockSpec((bm, DO), lambda i: (i, 0)),
        out_shape=jax.ShapeDtypeStruct((N, DO), jnp.float32),
    )(p, b2p)


def kernel(edge_index, edge_weight, feature, W1, b1, W2, b2):
    pad = E_PAD - E
    ei = jnp.pad(edge_index.astype(jnp.int32), ((0, 0), (0, pad)))
    w = jnp.pad(edge_weight.astype(jnp.float32), (0, pad))
    src2 = ei[0].reshape(NW * NSUB, SUB)
    dst2 = ei[1].reshape(NW * NSUB, SUB)

    mask2 = jnp.asarray(_MASK2)
    W2p = jnp.concatenate(
        [W2.astype(jnp.float32), jnp.zeros((DH, 16 - DO), jnp.float32)], axis=1)
    b2p = jnp.concatenate(
        [b2.astype(jnp.float32), jnp.zeros((16 - DO,), jnp.float32)]
    ).reshape(1, 16)

    s1 = _tc1(feature, W1)
    p1 = _spmm32(src2, dst2, w, s1, jnp.zeros((N, DH), jnp.float32))
    s2 = _tc2(p1, b1.astype(jnp.float32).reshape(1, DH), mask2, W2p)
    p2 = _spmm16(src2, dst2, w, s2, jnp.zeros((N, 16), jnp.float32))
    return _tc3(p2, b2p)


# speedup vs baseline: 1.0119x; 1.0119x over previous
import functools

import jax
import jax.numpy as jnp
import numpy as np
from jax import lax
from jax.experimental import pallas as pl
from jax.experimental.pallas import tpu as pltpu
from jax.experimental.pallas import tpu_sc as plsc

N = 10000
E = 160000
DIN = 256
DH = 32
DO = 7

NC = 2
NS = 16
NW = NC * NS
SUB = 128
EPW = 5120
NSUB = EPW // SUB
G = 4
NG = NSUB // G
E_PAD = NW * EPW
RPT = 624
TAIL = N - NS * RPT


def _threefry2x32(k0, k1, x0, x1):
    R0 = (13, 15, 26, 6)
    R1 = (17, 29, 16, 24)
    ks = (k0, k1, np.uint32(k0 ^ k1 ^ np.uint32(0x1BD11BDA)))
    x0 = (x0 + ks[0]).astype(np.uint32)
    x1 = (x1 + ks[1]).astype(np.uint32)
    for d in range(5):
        for r in (R0 if d % 2 == 0 else R1):
            x0 = (x0 + x1).astype(np.uint32)
            x1 = ((x1 << np.uint32(r)) | (x1 >> np.uint32(32 - r))).astype(np.uint32)
            x1 = x1 ^ x0
        x0 = (x0 + ks[(d + 1) % 3]).astype(np.uint32)
        x1 = (x1 + ks[(d + 2) % 3] + np.uint32(d + 1)).astype(np.uint32)
    return x0, x1


def _bernoulli_mask(seed, shape):
    n = int(np.prod(shape))
    x0, x1 = _threefry2x32(np.uint32(0), np.uint32(seed),
                           np.zeros(n, np.uint32),
                           np.arange(n, dtype=np.uint32))
    bits = x0 ^ x1
    floats = ((bits >> np.uint32(9)) | np.uint32(0x3F800000)).view(np.float32) - 1.0
    return (floats < 0.5).reshape(shape)


_MASK2 = _bernoulli_mask(42, (N, DH)).astype(np.float32) * 2.0


def _make_spmm(D):
    mesh = plsc.VectorSubcoreMesh(core_axis_name="c", subcore_axis_name="s")

    @functools.partial(
        pl.kernel,
        mesh=mesh,
        compiler_params=pltpu.CompilerParams(use_tc_tiling_on_sc=False),
        out_type=jax.ShapeDtypeStruct((NC, N, D), jnp.float32),
        scratch_types=[
            pltpu.VMEM((NSUB, SUB), jnp.int32),
            pltpu.VMEM((NSUB, SUB), jnp.int32),
            pltpu.VMEM((EPW,), jnp.float32),
            pltpu.VMEM((2, G * SUB, D), jnp.float32),
            pltpu.VMEM_SHARED((N, D), jnp.float32),
            pltpu.VMEM_SHARED((N, D), jnp.float32),
            pltpu.SemaphoreType.DMA,
        ],
    )
    def spmm(src_hbm, dst_hbm, w_hbm, x_hbm, zeros_hbm, out_hbm,
             src_v, dst_v, w_v, rows_v, acc_sh, x_sh, sem):
        cid = lax.axis_index("c")
        sid = lax.axis_index("s")
        wid = cid * NS + sid

        stage = [
            pltpu.async_copy(zeros_hbm.at[pl.ds(sid * RPT, RPT)],
                             acc_sh.at[pl.ds(sid * RPT, RPT)], sem),
            pltpu.async_copy(x_hbm.at[pl.ds(sid * RPT, RPT)],
                             x_sh.at[pl.ds(sid * RPT, RPT)], sem),
            pltpu.async_copy(src_hbm.at[pl.ds(wid * NSUB, NSUB)], src_v, sem),
            pltpu.async_copy(dst_hbm.at[pl.ds(wid * NSUB, NSUB)], dst_v, sem),
            pltpu.async_copy(w_hbm.at[pl.ds(wid * EPW, EPW)], w_v, sem),
        ]

        @pl.when(sid == NS - 1)
        def _():
            pltpu.async_copy(zeros_hbm.at[pl.ds(NS * RPT, TAIL)],
                             acc_sh.at[pl.ds(NS * RPT, TAIL)], sem).wait()
            pltpu.async_copy(x_hbm.at[pl.ds(NS * RPT, TAIL)],
                             x_sh.at[pl.ds(NS * RPT, TAIL)], sem).wait()

        for c in stage:
            c.wait()

        plsc.subcore_barrier()

        def fire(g, b):
            for k in range(G):
                pltpu.async_copy(
                    x_sh.at[src_v.at[g * G + k]],
                    rows_v.at[b, pl.ds(k * SUB, SUB)], sem)

        def drain(g, b):
            for k in range(G):
                pltpu.make_async_copy(
                    x_sh.at[src_v.at[g * G + k]],
                    rows_v.at[b, pl.ds(k * SUB, SUB)], sem).wait()

        def process(g, b):
            for k in range(G):
                j = g * G + k

                def scale(q, c, _k=k):
                    w16 = w_v[pl.ds((g * G + _k) * SUB + q * 16, 16)]
                    for e in range(16):
                        wb = jnp.full((16,), w16[e], jnp.float32)
                        r = _k * SUB + q * 16 + e
                        for d in range(D // 16):
                            v = rows_v[b, r, pl.ds(d * 16, 16)]
                            rows_v[b, r, pl.ds(d * 16, 16)] = v * wb
                    return c

                lax.fori_loop(0, SUB // 16, scale, 0, unroll=4)
                pltpu.sync_copy(rows_v.at[b, pl.ds(k * SUB, SUB)],
                                acc_sh.at[dst_v.at[j]], add=True)

        fire(0, 0)

        def pipe(t, carry):
            g0 = 2 * t
            fire(g0 + 1, 1)
            drain(g0, 0)
            process(g0, 0)

            @pl.when(g0 + 2 < NG)
            def _():
                fire(g0 + 2, 0)

            drain(g0 + 1, 1)
            process(g0 + 1, 1)
            return carry

        lax.fori_loop(0, NG // 2, pipe, 0)

        plsc.subcore_barrier()

        pltpu.sync_copy(acc_sh.at[pl.ds(sid * RPT, RPT)],
                        out_hbm.at[cid, pl.ds(sid * RPT, RPT)])

        @pl.when(sid == NS - 1)
        def _():
            pltpu.sync_copy(acc_sh.at[pl.ds(NS * RPT, TAIL)],
                            out_hbm.at[cid, pl.ds(NS * RPT, TAIL)])

    return spmm


_spmm32 = _make_spmm(DH)
_spmm16 = _make_spmm(16)


def _mm1_body(x_ref, w_ref, o_ref):
    o_ref[...] = jnp.dot(x_ref[...], w_ref[...],
                         preferred_element_type=jnp.float32)


def _tc1(x, w):
    bm = 2000
    return pl.pallas_call(
        _mm1_body,
        grid=(N // bm,),
        in_specs=[
            pl.BlockSpec((bm, DIN), lambda i: (i, 0)),
            pl.BlockSpec((DIN, DH), lambda i: (0, 0)),
        ],
        out_specs=pl.BlockSpec((bm, DH), lambda i: (i, 0)),
        out_shape=jax.ShapeDtypeStruct((N, DH), jnp.float32),
    )(x, w)


def _tc2_body(p_ref, b1_ref, m_ref, w2_ref, o_ref):
    h = p_ref[0] + p_ref[1] + b1_ref[...]
    h = jnp.maximum(h, 0.0) * m_ref[...]
    o_ref[...] = jnp.dot(h, w2_ref[...], preferred_element_type=jnp.float32)


def _tc2(p, b1, mask2, w2p):
    bm = 2000
    return pl.pallas_call(
        _tc2_body,
        grid=(N // bm,),
        in_specs=[
            pl.BlockSpec((NC, bm, DH), lambda i: (0, i, 0)),
            pl.BlockSpec((1, DH), lambda i: (0, 0)),
            pl.BlockSpec((bm, DH), lambda i: (i, 0)),
            pl.BlockSpec((DH, 16), lambda i: (0, 0)),
        ],
        out_specs=pl.BlockSpec((bm, 16), lambda i: (i, 0)),
        out_shape=jax.ShapeDtypeStruct((N, 16), jnp.float32),
    )(p, b1, mask2, w2p)


def _tc3_body(p_ref, b2_ref, o_ref):
    s = p_ref[0] + p_ref[1] + b2_ref[...]
    o_ref[...] = s[:, :DO]


def _tc3(p, b2p):
    bm = 2000
    return pl.pallas_call(
        _tc3_body,
        grid=(N // bm,),
        in_specs=[
            pl.BlockSpec((NC, bm, 16), lambda i: (0, i, 0)),
            pl.BlockSpec((1, 16), lambda i: (0, 0)),
        ],
        out_specs=pl.BlockSpec((bm, DO), lambda i: (i, 0)),
        out_shape=jax.ShapeDtypeStruct((N, DO), jnp.float32),
    )(p, b2p)


def kernel(edge_index, edge_weight, feature, W1, b1, W2, b2):
    pad = E_PAD - E
    ei = jnp.pad(edge_index.astype(jnp.int32), ((0, 0), (0, pad)))
    w = jnp.pad(edge_weight.astype(jnp.float32), (0, pad))
    src2 = ei[0].reshape(NW * NSUB, SUB)
    dst2 = ei[1].reshape(NW * NSUB, SUB)

    mask2 = jnp.asarray(_MASK2)
    W2p = jnp.concatenate(
        [W2.astype(jnp.float32), jnp.zeros((DH, 16 - DO), jnp.float32)], axis=1)
    b2p = jnp.concatenate(
        [b2.astype(jnp.float32), jnp.zeros((16 - DO,), jnp.float32)]
    ).reshape(1, 16)

    s1 = _tc1(feature, W1)
    p1 = _spmm32(src2, dst2, w, s1, jnp.zeros((N, DH), jnp.float32))
    s2 = _tc2(p1, b1.astype(jnp.float32).reshape(1, DH), mask2, W2p)
    p2 = _spmm16(src2, dst2, w, s2, jnp.zeros((N, 16), jnp.float32))
    return _tc3(p2, b2p)
